# baseline (speedup 1.0000x reference)
import jax
import jax.numpy as jnp
from jax import lax
from jax.experimental import pallas as pl
from jax.experimental.pallas import tpu as pltpu
from jax.experimental.pallas import tpu_sc as plsc

B = 65536
N_ATOMS = 51
V_MIN = -10.0
V_MAX = 10.0
GAMMA = 0.99

P_OFF = 0
LG_OFF = 64
OV_LANE = 120

NC = 2
NS = 16
LANES = 16
NW = NC * NS
ROWS_PER_W = B // NW
CHUNK = 128
N_CHUNKS = ROWS_PER_W // CHUNK
GROUPS = CHUNK // LANES

TC_R = 2048


def _tc_pack_body(old_ref, next_ref, atoms_ref, pk_ref):
    old = old_ref[...]
    logc = jnp.log(jnp.clip(old, 1e-5, 1.0 - 1e-5))
    ov = jnp.sum(old * atoms_ref[...], axis=1, keepdims=True)
    p128 = jnp.pad(next_ref[...], ((0, 0), (P_OFF, 128 - P_OFF - N_ATOMS)))
    l128 = jnp.pad(logc, ((0, 0), (LG_OFF, 128 - LG_OFF - N_ATOMS)))
    lane = lax.broadcasted_iota(jnp.int32, (TC_R, 128), 1)
    ovm = jnp.where(lane == OV_LANE, jnp.broadcast_to(ov, (TC_R, 128)), 0.0)
    pk_ref[...] = p128 + l128 + ovm


def _tc_pack(old_pmfs, next_pmfs, atoms2d):
    return pl.pallas_call(
        _tc_pack_body,
        grid=(B // TC_R,),
        in_specs=[
            pl.BlockSpec((TC_R, N_ATOMS), lambda m: (m, 0)),
            pl.BlockSpec((TC_R, N_ATOMS), lambda m: (m, 0)),
            pl.BlockSpec((1, N_ATOMS), lambda m: (0, 0)),
        ],
        out_specs=pl.BlockSpec((TC_R, 128), lambda m: (m, 0)),
        out_shape=jax.ShapeDtypeStruct((B, 128), jnp.float32),
    )(old_pmfs, next_pmfs, atoms2d)


PKW = 129


def _sc_loss_body(pk_hbm, r_hbm, d_hbm, cv_hbm, ov_hbm, part_hbm,
                  pk0, pk1, r_buf, d_buf, cv_buf, ov_buf, acc_buf,
                  spk0, spk1):
    pk_bufs = (pk0, pk1)
    spks = (spk0, spk1)
    wid = lax.axis_index("c") * NS + lax.axis_index("s")
    iota = lax.iota(jnp.int32, LANES)
    zero_i = jnp.zeros((LANES,), jnp.int32)
    ov_col = jnp.full((LANES,), OV_LANE, jnp.int32)

    pltpu.sync_copy(cv_hbm, cv_buf)
    s0 = cv_buf[pl.ds(0, LANES)]
    s1 = cv_buf[pl.ds(LANES, LANES)]
    s2 = cv_buf[pl.ds(2 * LANES, LANES)]

    def issue(c, slot):
        row0 = wid * ROWS_PER_W + c * CHUNK
        return (
            pltpu.async_copy(pk_hbm.at[pl.ds(row0, CHUNK), :],
                             pk_bufs[slot].at[:, pl.ds(0, 128)], spks[slot]),
        )

    def one_j(pk_buf, nb, base, jcol):
        bb = jnp.minimum(jnp.maximum(nb, float(LG_OFF)),
                         float(LG_OFF + N_ATOMS - 1))
        li = bb.astype(jnp.int32)
        frac = bb - li.astype(jnp.float32)
        ui = jnp.minimum(li + 1, LG_OFF + N_ATOMS - 1)
        pv = plsc.load_gather(pk_buf, [base, jcol])
        ll = plsc.load_gather(pk_buf, [base, li])
        lu = plsc.load_gather(pk_buf, [base, ui])
        return pv * ((1.0 - frac) * ll + frac * lu)

    pend = [issue(0, 0), None]
    total = jnp.zeros((LANES,), jnp.float32)
    for c in range(N_CHUNKS):
        slot = c & 1
        if c + 1 < N_CHUNKS:
            pend[1 - slot] = issue(c + 1, 1 - slot)
        row0 = wid * ROWS_PER_W + c * CHUNK
        pltpu.sync_copy(r_hbm.at[pl.ds(row0, CHUNK), :], r_buf)
        pltpu.sync_copy(d_hbm.at[pl.ds(row0, CHUNK), :], d_buf)
        for h in pend[slot]:
            h.wait()
        pk_buf = pk_bufs[slot]

        def group_body(g, acc_g, pk_buf=pk_buf, r_buf=r_buf, d_buf=d_buf):
            base = g * LANES + iota
            rv = plsc.load_gather(r_buf, [base, zero_i])
            dv = plsc.load_gather(d_buf, [base, zero_i])
            omd = 1.0 - dv
            a2 = (rv - V_MIN) * s0 + s1 * omd + float(LG_OFF)
            b2 = s2 * omd
            ovv = plsc.load_gather(pk_buf, [base, ov_col])
            plsc.store_scatter(ov_buf, [base], ovv)

            @plsc.parallel_loop(0, N_ATOMS, 3, carry=acc_g)
            def jloop(j, acc):
                jj = jnp.full((LANES,), j, jnp.int32)
                jf = jj.astype(jnp.float32)
                nb0 = a2 + b2 * jf
                nb1 = nb0 + b2
                nb2 = nb1 + b2
                c0 = one_j(pk_buf, nb0, base, jj)
                c1 = one_j(pk_buf, nb1, base, jj + 1)
                c2 = one_j(pk_buf, nb2, base, jj + 2)
                return acc + ((c0 + c1) + c2)
            return jloop
        total = lax.fori_loop(0, GROUPS, group_body, total)

        pltpu.sync_copy(ov_buf, ov_hbm.at[pl.ds(row0, CHUNK)])

    acc_buf[...] = total
    pltpu.sync_copy(acc_buf, part_hbm.at[pl.ds(wid * LANES, LANES)])


def _sc_loss(pk, rewards, dones, cvec):
    run = pl.kernel(
        _sc_loss_body,
        out_type=[
            jax.ShapeDtypeStruct((B,), jnp.float32),
            jax.ShapeDtypeStruct((NW * LANES,), jnp.float32),
        ],
        mesh=plsc.VectorSubcoreMesh(core_axis_name="c", subcore_axis_name="s"),
        compiler_params=pltpu.CompilerParams(needs_layout_passes=False),
        scratch_types=[
            pltpu.VMEM((CHUNK, PKW), jnp.float32),
            pltpu.VMEM((CHUNK, PKW), jnp.float32),
            pltpu.VMEM((CHUNK, 1), jnp.float32),
            pltpu.VMEM((CHUNK, 1), jnp.float32),
            pltpu.VMEM((64,), jnp.float32),
            pltpu.VMEM((CHUNK,), jnp.float32),
            pltpu.VMEM((LANES,), jnp.float32),
            pltpu.SemaphoreType.DMA,
            pltpu.SemaphoreType.DMA,
        ],
    )
    return run(pk, rewards, dones, cvec)


def kernel(next_pmfs, rewards, dones, old_pmfs, atoms):
    dz = atoms[1] - atoms[0]
    inv_dz = 1.0 / dz
    s0 = jnp.full((LANES,), inv_dz, jnp.float32)
    s1 = jnp.full((LANES,), GAMMA * V_MIN * inv_dz, jnp.float32)
    s2 = jnp.full((LANES,), GAMMA * dz * inv_dz, jnp.float32)
    cvec = jnp.concatenate([s0, s1, s2, jnp.zeros((LANES,), jnp.float32)])
    pk = _tc_pack(old_pmfs, next_pmfs, atoms.reshape(1, N_ATOMS))
    old_val, parts = _sc_loss(pk, rewards, dones, cvec)
    loss = -(jnp.sum(parts) / B)
    return (old_val, loss)


# speedup vs baseline: 1.0465x; 1.0465x over previous
import jax
import jax.numpy as jnp
from jax import lax
from jax.experimental import pallas as pl
from jax.experimental.pallas import tpu as pltpu
from jax.experimental.pallas import tpu_sc as plsc

B = 65536
N_ATOMS = 51
V_MIN = -10.0
V_MAX = 10.0
GAMMA = 0.99

NC = 2
NS = 16
LANES = 16
NW = NC * NS
ROWS_PER_W = B // NW
CHUNK = 256
N_CHUNKS = ROWS_PER_W // CHUNK
GROUPS = CHUNK // LANES
ZSTEPS = CHUNK * N_ATOMS // LANES


def _sc_project_body(p_hbm, r_hbm, d_hbm, cv_hbm, t_hbm,
                     p0, p1, t_buf, r_buf, d_buf, cv_buf, sp0, sp1):
    p_bufs = (p0, p1)
    sps = (sp0, sp1)
    wid = lax.axis_index("c") * NS + lax.axis_index("s")
    iota = lax.iota(jnp.int32, LANES)
    zeros16 = jnp.zeros((LANES,), jnp.float32)
    zero_i = jnp.zeros((LANES,), jnp.int32)

    pltpu.sync_copy(cv_hbm, cv_buf)
    s0 = cv_buf[pl.ds(0, LANES)]
    s1 = cv_buf[pl.ds(LANES, LANES)]
    s2 = cv_buf[pl.ds(2 * LANES, LANES)]

    def issue(c, slot):
        row0 = wid * ROWS_PER_W + c * CHUNK
        return pltpu.async_copy(
            p_hbm.at[pl.ds(row0 * N_ATOMS, CHUNK * N_ATOMS)],
            p_bufs[slot], sps[slot])

    pend = [issue(0, 0), None]
    for c in range(N_CHUNKS):
        slot = c & 1
        if c + 1 < N_CHUNKS:
            pend[1 - slot] = issue(c + 1, 1 - slot)
        row0 = wid * ROWS_PER_W + c * CHUNK
        pltpu.sync_copy(r_hbm.at[pl.ds(row0, CHUNK), :], r_buf)
        pltpu.sync_copy(d_hbm.at[pl.ds(row0, CHUNK), :], d_buf)

        @plsc.parallel_loop(0, ZSTEPS, unroll=8)
        def zbody(i):
            plsc.store_scatter(t_buf, [i * LANES + iota], zeros16)

        pend[slot].wait()
        p_buf = p_bufs[slot]

        def group_body(g, carry, p_buf=p_buf):
            base = g * LANES + iota
            rv = plsc.load_gather(r_buf, [base, zero_i])
            dv = plsc.load_gather(d_buf, [base, zero_i])
            omd = 1.0 - dv
            a2 = (rv - V_MIN) * s0 + s1 * omd
            b2 = s2 * omd
            rl51 = base * N_ATOMS

            @plsc.parallel_loop(0, N_ATOMS, unroll=3)
            def jbody(j):
                jj = jnp.full((LANES,), j, jnp.int32)
                jf = jj.astype(jnp.float32)
                nb = a2 + b2 * jf
                bb = jnp.minimum(jnp.maximum(nb, 0.0), float(N_ATOMS - 1))
                li = bb.astype(jnp.int32)
                frac = bb - li.astype(jnp.float32)
                ui = jnp.minimum(li + 1, N_ATOMS - 1)
                pv = plsc.load_gather(p_buf, [rl51 + jj])
                plsc.addupdate_scatter(t_buf, [rl51 + li], (1.0 - frac) * pv)
                plsc.addupdate_scatter(t_buf, [rl51 + ui], frac * pv)
            return carry
        lax.fori_loop(0, GROUPS, group_body, 0)

        pltpu.sync_copy(t_buf, t_hbm.at[pl.ds(row0 * N_ATOMS, CHUNK * N_ATOMS)])


def _sc_project(p_flat, rewards, dones, cvec):
    run = pl.kernel(
        _sc_project_body,
        out_type=jax.ShapeDtypeStruct((B * N_ATOMS,), jnp.float32),
        mesh=plsc.VectorSubcoreMesh(core_axis_name="c", subcore_axis_name="s"),
        compiler_params=pltpu.CompilerParams(needs_layout_passes=False),
        scratch_types=[
            pltpu.VMEM((CHUNK * N_ATOMS,), jnp.float32),
            pltpu.VMEM((CHUNK * N_ATOMS,), jnp.float32),
            pltpu.VMEM((CHUNK * N_ATOMS,), jnp.float32),
            pltpu.VMEM((CHUNK, 1), jnp.float32),
            pltpu.VMEM((CHUNK, 1), jnp.float32),
            pltpu.VMEM((64,), jnp.float32),
            pltpu.SemaphoreType.DMA,
            pltpu.SemaphoreType.DMA,
        ],
    )
    return run(p_flat, rewards, dones, cvec)


TC_R = 2048


def _tc_loss_body(t_ref, old_ref, atoms_ref, oldval_ref, loss_ref, acc_ref):
    m = pl.program_id(0)
    old = old_ref[...]
    logc = jnp.log(jnp.clip(old, 1e-5, 1.0 - 1e-5))
    part = jnp.sum(t_ref[...] * logc)
    prev = jnp.where(m == 0, 0.0, acc_ref[0])
    acc = prev + part
    acc_ref[0] = acc
    oldval_ref[...] = jnp.sum(old * atoms_ref[...], axis=1, keepdims=True)

    @pl.when(m == pl.num_programs(0) - 1)
    def _():
        loss_ref[0] = -acc / B


def _tc_loss(target, old_pmfs, atoms2d):
    return pl.pallas_call(
        _tc_loss_body,
        grid=(B // TC_R,),
        in_specs=[
            pl.BlockSpec((TC_R, N_ATOMS), lambda m: (m, 0)),
            pl.BlockSpec((TC_R, N_ATOMS), lambda m: (m, 0)),
            pl.BlockSpec((1, N_ATOMS), lambda m: (0, 0)),
        ],
        out_specs=[
            pl.BlockSpec((TC_R, 1), lambda m: (m, 0)),
            pl.BlockSpec(memory_space=pltpu.SMEM, block_shape=(1,),
                         index_map=lambda m: (0,)),
        ],
        out_shape=[
            jax.ShapeDtypeStruct((B, 1), jnp.float32),
            jax.ShapeDtypeStruct((1,), jnp.float32),
        ],
        scratch_shapes=[pltpu.SMEM((1,), jnp.float32)],
    )(target, old_pmfs, atoms2d)


def kernel(next_pmfs, rewards, dones, old_pmfs, atoms):
    dz = atoms[1] - atoms[0]
    inv_dz = 1.0 / dz
    s0 = jnp.full((LANES,), inv_dz, jnp.float32)
    s1 = jnp.full((LANES,), GAMMA * V_MIN * inv_dz, jnp.float32)
    s2 = jnp.full((LANES,), GAMMA * dz * inv_dz, jnp.float32)
    cvec = jnp.concatenate([s0, s1, s2, jnp.zeros((LANES,), jnp.float32)])
    t_flat = _sc_project(next_pmfs.reshape(-1), rewards, dones, cvec)
    target = t_flat.reshape(B, N_ATOMS)
    old_val2d, loss1 = _tc_loss(target, old_pmfs, atoms.reshape(1, N_ATOMS))
    return (old_val2d.reshape(B), loss1.reshape(()))


# speedup vs baseline: 1.3431x; 1.2835x over previous
import jax
import jax.numpy as jnp
from jax import lax
from jax.experimental import pallas as pl
from jax.experimental.pallas import tpu as pltpu
from jax.experimental.pallas import tpu_sc as plsc

B = 65536
N_ATOMS = 51
V_MIN = -10.0
V_MAX = 10.0
GAMMA = 0.99

NC = 2
NS = 16
LANES = 16
NW = NC * NS
ROWS_PER_W = B // NW
CHUNK = 512
N_CHUNKS = ROWS_PER_W // CHUNK
GROUPS = CHUNK // LANES
ZSTEPS = CHUNK * N_ATOMS // LANES


def _sc_project_body(p_hbm, r_hbm, d_hbm, cv_hbm, t_hbm,
                     p0, p1, t_buf, r_buf, d_buf, cv_buf, sp0, sp1):
    p_bufs = (p0, p1)
    sps = (sp0, sp1)
    wid = lax.axis_index("c") * NS + lax.axis_index("s")
    iota = lax.iota(jnp.int32, LANES)
    zeros16 = jnp.zeros((LANES,), jnp.float32)
    zero_i = jnp.zeros((LANES,), jnp.int32)

    pltpu.sync_copy(cv_hbm, cv_buf)
    s0 = cv_buf[pl.ds(0, LANES)]
    s1 = cv_buf[pl.ds(LANES, LANES)]
    s2 = cv_buf[pl.ds(2 * LANES, LANES)]

    def issue(c, slot):
        row0 = wid * ROWS_PER_W + c * CHUNK
        return pltpu.async_copy(
            p_hbm.at[pl.ds(row0 * N_ATOMS, CHUNK * N_ATOMS)],
            p_bufs[slot], sps[slot])

    pend = [issue(0, 0), None]
    for c in range(N_CHUNKS):
        slot = c & 1
        if c + 1 < N_CHUNKS:
            pend[1 - slot] = issue(c + 1, 1 - slot)
        row0 = wid * ROWS_PER_W + c * CHUNK
        pltpu.sync_copy(r_hbm.at[pl.ds(row0, CHUNK)], r_buf)
        pltpu.sync_copy(d_hbm.at[pl.ds(row0, CHUNK)], d_buf)

        @plsc.parallel_loop(0, ZSTEPS, unroll=8)
        def zbody(i):
            plsc.store_scatter(t_buf, [i * LANES + iota], zeros16)

        pend[slot].wait()
        p_buf = p_bufs[slot]

        def group_body(g, carry, p_buf=p_buf):
            base = g * LANES + iota
            rv = plsc.load_gather(r_buf, [base])
            dv = plsc.load_gather(d_buf, [base])
            omd = 1.0 - dv
            a2 = (rv - V_MIN) * s0 + s1 * omd
            b2 = s2 * omd
            rl51 = base * N_ATOMS

            @plsc.parallel_loop(0, N_ATOMS, unroll=3)
            def jbody(j):
                jj = jnp.full((LANES,), j, jnp.int32)
                jf = jj.astype(jnp.float32)
                nb = a2 + b2 * jf
                bb = jnp.minimum(jnp.maximum(nb, 0.0), float(N_ATOMS - 1))
                li = bb.astype(jnp.int32)
                frac = bb - li.astype(jnp.float32)
                ui = jnp.minimum(li + 1, N_ATOMS - 1)
                pv = plsc.load_gather(p_buf, [rl51 + jj])
                plsc.addupdate_scatter(t_buf, [rl51 + li], (1.0 - frac) * pv)
                plsc.addupdate_scatter(t_buf, [rl51 + ui], frac * pv)
            return carry
        lax.fori_loop(0, GROUPS, group_body, 0)

        pltpu.sync_copy(t_buf, t_hbm.at[pl.ds(row0 * N_ATOMS, CHUNK * N_ATOMS)])


def _sc_project(p_flat, rewards, dones, cvec):
    run = pl.kernel(
        _sc_project_body,
        out_type=jax.ShapeDtypeStruct((B * N_ATOMS,), jnp.float32),
        mesh=plsc.VectorSubcoreMesh(core_axis_name="c", subcore_axis_name="s"),
        compiler_params=pltpu.CompilerParams(needs_layout_passes=False),
        scratch_types=[
            pltpu.VMEM((CHUNK * N_ATOMS,), jnp.float32),
            pltpu.VMEM((CHUNK * N_ATOMS,), jnp.float32),
            pltpu.VMEM((CHUNK * N_ATOMS,), jnp.float32),
            pltpu.VMEM((CHUNK,), jnp.float32),
            pltpu.VMEM((CHUNK,), jnp.float32),
            pltpu.VMEM((64,), jnp.float32),
            pltpu.SemaphoreType.DMA,
            pltpu.SemaphoreType.DMA,
        ],
    )
    return run(p_flat, rewards, dones, cvec)


TC_R = 2048


def _tc_loss_body(t_ref, old_ref, atoms_ref, oldval_ref, loss_ref, acc_ref):
    m = pl.program_id(0)
    old = old_ref[...]
    logc = jnp.log(jnp.clip(old, 1e-5, 1.0 - 1e-5))
    part = jnp.sum(t_ref[...] * logc)
    prev = jnp.where(m == 0, 0.0, acc_ref[0])
    acc = prev + part
    acc_ref[0] = acc
    oldval_ref[...] = jnp.sum(old * atoms_ref[...], axis=1, keepdims=True)

    @pl.when(m == pl.num_programs(0) - 1)
    def _():
        loss_ref[0] = -acc / B


def _tc_loss(target, old_pmfs, atoms2d):
    return pl.pallas_call(
        _tc_loss_body,
        grid=(B // TC_R,),
        in_specs=[
            pl.BlockSpec((TC_R, N_ATOMS), lambda m: (m, 0)),
            pl.BlockSpec((TC_R, N_ATOMS), lambda m: (m, 0)),
            pl.BlockSpec((1, N_ATOMS), lambda m: (0, 0)),
        ],
        out_specs=[
            pl.BlockSpec((TC_R, 1), lambda m: (m, 0)),
            pl.BlockSpec(memory_space=pltpu.SMEM, block_shape=(1,),
                         index_map=lambda m: (0,)),
        ],
        out_shape=[
            jax.ShapeDtypeStruct((B, 1), jnp.float32),
            jax.ShapeDtypeStruct((1,), jnp.float32),
        ],
        scratch_shapes=[pltpu.SMEM((1,), jnp.float32)],
    )(target, old_pmfs, atoms2d)


def kernel(next_pmfs, rewards, dones, old_pmfs, atoms):
    dz = atoms[1] - atoms[0]
    inv_dz = 1.0 / dz
    s0 = jnp.full((LANES,), inv_dz, jnp.float32)
    s1 = jnp.full((LANES,), GAMMA * V_MIN * inv_dz, jnp.float32)
    s2 = jnp.full((LANES,), GAMMA * dz * inv_dz, jnp.float32)
    cvec = jnp.concatenate([s0, s1, s2, jnp.zeros((LANES,), jnp.float32)])
    t_flat = _sc_project(next_pmfs.reshape(-1), rewards.reshape(-1),
                         dones.reshape(-1), cvec)
    target = t_flat.reshape(B, N_ATOMS)
    old_val2d, loss1 = _tc_loss(target, old_pmfs, atoms.reshape(1, N_ATOMS))
    return (old_val2d.reshape(B), loss1.reshape(()))


# speedup vs baseline: 1.3579x; 1.0110x over previous
import jax
import jax.numpy as jnp
from jax import lax
from jax.experimental import pallas as pl
from jax.experimental.pallas import tpu as pltpu
from jax.experimental.pallas import tpu_sc as plsc

B = 65536
N_ATOMS = 51
V_MIN = -10.0
V_MAX = 10.0
GAMMA = 0.99

NC = 2
NS = 16
LANES = 16
NW = NC * NS
ROWS_PER_W = B // NW
CHUNK = 512
N_CHUNKS = ROWS_PER_W // CHUNK
GROUPS = CHUNK // LANES
ZSTEPS = CHUNK * N_ATOMS // LANES


def _sc_project_body(p_hbm, r_hbm, d_hbm, cv_hbm, t_hbm,
                     p0, p1, t0, t1, r_buf, d_buf, cv_buf,
                     sp0, sp1, st0, st1):
    p_bufs = (p0, p1)
    t_bufs = (t0, t1)
    sps = (sp0, sp1)
    sts = (st0, st1)
    wid = lax.axis_index("c") * NS + lax.axis_index("s")
    iota = lax.iota(jnp.int32, LANES)
    zeros16 = jnp.zeros((LANES,), jnp.float32)
    zero_i = jnp.zeros((LANES,), jnp.int32)

    pltpu.sync_copy(cv_hbm, cv_buf)
    s0 = cv_buf[pl.ds(0, LANES)]
    s1 = cv_buf[pl.ds(LANES, LANES)]
    s2 = cv_buf[pl.ds(2 * LANES, LANES)]

    def issue(c, slot):
        row0 = wid * ROWS_PER_W + c * CHUNK
        return pltpu.async_copy(
            p_hbm.at[pl.ds(row0 * N_ATOMS, CHUNK * N_ATOMS)],
            p_bufs[slot], sps[slot])

    pend = [issue(0, 0), None]
    pend_out = [None, None]
    for c in range(N_CHUNKS):
        slot = c & 1
        if c + 1 < N_CHUNKS:
            pend[1 - slot] = issue(c + 1, 1 - slot)
        row0 = wid * ROWS_PER_W + c * CHUNK
        pltpu.sync_copy(r_hbm.at[pl.ds(row0, CHUNK)], r_buf)
        pltpu.sync_copy(d_hbm.at[pl.ds(row0, CHUNK)], d_buf)
        if pend_out[slot] is not None:
            pend_out[slot].wait()
        t_buf = t_bufs[slot]

        @plsc.parallel_loop(0, ZSTEPS, unroll=8)
        def zbody(i, t_buf=t_buf):
            plsc.store_scatter(t_buf, [i * LANES + iota], zeros16)

        pend[slot].wait()
        p_buf = p_bufs[slot]

        def group_body(g, carry, p_buf=p_buf, t_buf=t_buf):
            base = g * LANES + iota
            rv = plsc.load_gather(r_buf, [base])
            dv = plsc.load_gather(d_buf, [base])
            omd = 1.0 - dv
            a2 = (rv - V_MIN) * s0 + s1 * omd
            b2 = s2 * omd
            rl51 = base * N_ATOMS

            @plsc.parallel_loop(0, N_ATOMS, unroll=3)
            def jbody(j):
                jj = jnp.full((LANES,), j, jnp.int32)
                jf = jj.astype(jnp.float32)
                nb = a2 + b2 * jf
                bb = jnp.minimum(jnp.maximum(nb, 0.0), float(N_ATOMS - 1))
                li = bb.astype(jnp.int32)
                frac = bb - li.astype(jnp.float32)
                ui = jnp.minimum(li + 1, N_ATOMS - 1)
                pv = plsc.load_gather(p_buf, [rl51 + jj])
                plsc.addupdate_scatter(t_buf, [rl51 + li], (1.0 - frac) * pv)
                plsc.addupdate_scatter(t_buf, [rl51 + ui], frac * pv)
            return carry
        lax.fori_loop(0, GROUPS, group_body, 0)

        pend_out[slot] = pltpu.async_copy(
            t_buf, t_hbm.at[pl.ds(row0 * N_ATOMS, CHUNK * N_ATOMS)], sts[slot])

    for po in pend_out:
        if po is not None:
            po.wait()


def _sc_project(p_flat, rewards, dones, cvec):
    run = pl.kernel(
        _sc_project_body,
        out_type=jax.ShapeDtypeStruct((B * N_ATOMS,), jnp.float32),
        mesh=plsc.VectorSubcoreMesh(core_axis_name="c", subcore_axis_name="s"),
        compiler_params=pltpu.CompilerParams(needs_layout_passes=False),
        scratch_types=[
            pltpu.VMEM((CHUNK * N_ATOMS,), jnp.float32),
            pltpu.VMEM((CHUNK * N_ATOMS,), jnp.float32),
            pltpu.VMEM((CHUNK * N_ATOMS,), jnp.float32),
            pltpu.VMEM((CHUNK * N_ATOMS,), jnp.float32),
            pltpu.VMEM((CHUNK,), jnp.float32),
            pltpu.VMEM((CHUNK,), jnp.float32),
            pltpu.VMEM((64,), jnp.float32),
            pltpu.SemaphoreType.DMA,
            pltpu.SemaphoreType.DMA,
            pltpu.SemaphoreType.DMA,
            pltpu.SemaphoreType.DMA,
        ],
    )
    return run(p_flat, rewards, dones, cvec)


TC_R = 2048


def _tc_loss_body(t_ref, old_ref, atoms_ref, oldval_ref, loss_ref, acc_ref):
    m = pl.program_id(0)
    old = old_ref[...]
    logc = jnp.log(jnp.clip(old, 1e-5, 1.0 - 1e-5))
    part = jnp.sum(t_ref[...] * logc)
    prev = jnp.where(m == 0, 0.0, acc_ref[0])
    acc = prev + part
    acc_ref[0] = acc
    oldval_ref[...] = jnp.sum(old * atoms_ref[...], axis=1, keepdims=True)

    @pl.when(m == pl.num_programs(0) - 1)
    def _():
        loss_ref[0] = -acc / B


def _tc_loss(target, old_pmfs, atoms2d):
    return pl.pallas_call(
        _tc_loss_body,
        grid=(B // TC_R,),
        in_specs=[
            pl.BlockSpec((TC_R, N_ATOMS), lambda m: (m, 0)),
            pl.BlockSpec((TC_R, N_ATOMS), lambda m: (m, 0)),
            pl.BlockSpec((1, N_ATOMS), lambda m: (0, 0)),
        ],
        out_specs=[
            pl.BlockSpec((TC_R, 1), lambda m: (m, 0)),
            pl.BlockSpec(memory_space=pltpu.SMEM, block_shape=(1,),
                         index_map=lambda m: (0,)),
        ],
        out_shape=[
            jax.ShapeDtypeStruct((B, 1), jnp.float32),
            jax.ShapeDtypeStruct((1,), jnp.float32),
        ],
        scratch_shapes=[pltpu.SMEM((1,), jnp.float32)],
    )(target, old_pmfs, atoms2d)


def kernel(next_pmfs, rewards, dones, old_pmfs, atoms):
    dz = atoms[1] - atoms[0]
    inv_dz = 1.0 / dz
    s0 = jnp.full((LANES,), inv_dz, jnp.float32)
    s1 = jnp.full((LANES,), GAMMA * V_MIN * inv_dz, jnp.float32)
    s2 = jnp.full((LANES,), GAMMA * dz * inv_dz, jnp.float32)
    cvec = jnp.concatenate([s0, s1, s2, jnp.zeros((LANES,), jnp.float32)])
    t_flat = _sc_project(next_pmfs.reshape(-1), rewards.reshape(-1),
                         dones.reshape(-1), cvec)
    target = t_flat.reshape(B, N_ATOMS)
    old_val2d, loss1 = _tc_loss(target, old_pmfs, atoms.reshape(1, N_ATOMS))
    return (old_val2d.reshape(B), loss1.reshape(()))
